# baseline (speedup 1.0000x reference)
import functools

import jax
import jax.numpy as jnp
from jax import lax
from jax.experimental import pallas as pl
from jax.experimental.pallas import tpu as pltpu
from jax.experimental.pallas import tpu_sc as plsc

D_MODEL = 2048
D_SAE = 32768
K = 64
B = 64
EPS = 1e-08
KB = K * B

F_BLK = 2048


def _preprocess_body(x_ref, diff_ref):
    x = x_ref[...]
    d0 = x[:, D_MODEL:] - x[:, :D_MODEL]
    mu = jnp.mean(d0, axis=0, keepdims=True)
    c = d0 - mu
    norms = jnp.sqrt(jnp.sum(c * c, axis=1, keepdims=True))
    scale = jnp.mean(norms)
    diff_ref[...] = c / (scale + EPS)


def _encode_body(diff_ref, w_ref, b_ref, acts_ref, scores_ref):
    w = w_ref[...]
    h = jnp.dot(diff_ref[...], w, preferred_element_type=jnp.float32,
                precision=jax.lax.Precision.DEFAULT)
    acts = jnp.maximum(h + b_ref[...], 0.0)
    norms = jnp.sqrt(jnp.sum(w * w, axis=0, keepdims=True))
    acts_ref[...] = acts
    scores_ref[...] = acts * norms


def _topk_body(scores_ref, acts_ref, sparse_ref, thresh_ref):
    scores = scores_ref[...]
    smax = jnp.max(scores)
    hi0 = jax.lax.bitcast_convert_type(smax, jnp.int32) + 1

    def step(_, carry):
        lo, hi = carry
        mid = lo + (hi - lo) // 2
        t = jax.lax.bitcast_convert_type(mid, jnp.float32)
        cnt = jnp.sum((scores >= t).astype(jnp.int32))
        big = cnt >= KB
        return (jnp.where(big, mid, lo), jnp.where(big, hi, mid))

    lo, _ = jax.lax.fori_loop(0, 31, step, (jnp.int32(0), hi0))
    t = jax.lax.bitcast_convert_type(lo, jnp.float32)
    thresh_ref[0, 0] = t
    sparse_ref[...] = jnp.where(scores >= t, acts_ref[...], 0.0)


L = 16
UNROLL = 8
G = 16
MAXNZ = K * B
ROWS_PER_W = 2


def _sc_decode_tec(sparse_hbm, wdec_hbm, out_hbm, row_v,
                   idx_v, val_v, gbuf_v, acc_v, sem, osem):
    info = plsc.get_sparse_core_info()
    nc = info.num_cores
    wid = lax.axis_index("s") * nc + lax.axis_index("c")
    lanes = lax.iota(jnp.int32, L)
    last_lane = jnp.full((L, 1), L - 1, jnp.int32)
    zero16 = jnp.zeros((L,), jnp.float32)

    def splat_last(x):
        return lax.gather(
            x, last_lane,
            lax.GatherDimensionNumbers(offset_dims=(),
                                       collapsed_slice_dims=(0,),
                                       start_index_map=(0,)),
            slice_sizes=(1,),
            mode=lax.GatherScatterMode.PROMISE_IN_BOUNDS)

    def do_row(r, _):
        row = wid * ROWS_PER_W + r
        pltpu.sync_copy(sparse_hbm.at[row], row_v)

        def scan_grp(gidx, curv):
            base = gidx * (UNROLL * L)
            for j in range(UNROLL):
                v = row_v[pl.ds(base + j * L, L)]
                m = v != 0.0
                inc = plsc.cumsum(m.astype(jnp.int32))
                pos = curv + inc - 1
                plsc.store_scatter(idx_v, [pos], base + j * L + lanes,
                                   mask=m)
                plsc.store_scatter(val_v, [pos], v, mask=m)
                curv = curv + plsc.all_reduce_population_count(m)
            return curv

        with jax.named_scope("sc_scan"):
            curv = lax.fori_loop(0, D_SAE // (UNROLL * L), scan_grp,
                                 jnp.zeros((L,), jnp.int32))
        cursor = curv[0]

        pad_base = (cursor // L) * L
        keep = lanes < (cursor - pad_base)
        val_v[pl.ds(pad_base, L)] = jnp.where(keep,
                                              val_v[pl.ds(pad_base, L)], 0.0)
        idx_v[pl.ds(pad_base, L)] = jnp.where(keep,
                                              idx_v[pl.ds(pad_base, L)], 0)

        for c in range(D_MODEL // L):
            acc_v[pl.ds(c * L, L)] = zero16

        nb = (cursor + G - 1) // G

        def start_gather(b, p):
            pltpu.make_async_copy(wdec_hbm.at[idx_v.at[pl.ds(b * G, G)]],
                                  gbuf_v.at[p], sem.at[p]).start()

        @pl.when(nb > 0)
        def _():
            start_gather(0, 0)

        def do_batch(b, _):
            p = lax.rem(b, 2)

            @pl.when(b + 1 < nb)
            def _():
                start_gather(b + 1, 1 - p)

            pltpu.make_async_copy(wdec_hbm.at[idx_v.at[pl.ds(b * G, G)]],
                                  gbuf_v.at[p], sem.at[p]).wait()
            vals = [plsc.load_gather(val_v,
                                     [jnp.full((L,), b * G + g, jnp.int32)])
                    for g in range(G)]

            def do_col(c, _):
                a = acc_v[pl.ds(c * L, L)]
                for g in range(G):
                    a = a + vals[g] * gbuf_v[p, g, pl.ds(c * L, L)]
                acc_v[pl.ds(c * L, L)] = a
                return 0

            lax.fori_loop(0, D_MODEL // L, do_col, 0)
            return 0

        with jax.named_scope("sc_gather"):
            lax.fori_loop(0, nb, do_batch, 0)
        pltpu.async_copy(acc_v, out_hbm.at[row], osem).wait()
        return 0

    lax.fori_loop(0, ROWS_PER_W, do_row, 0)


def _sc_decode(sparse, W_dec):
    mesh = plsc.VectorSubcoreMesh(core_axis_name="c", subcore_axis_name="s")
    return pl.kernel(
        _sc_decode_tec,
        out_type=jax.ShapeDtypeStruct((B, D_MODEL), jnp.float32),
        mesh=mesh,
        compiler_params=pltpu.CompilerParams(needs_layout_passes=False),
        scratch_types=[
            pltpu.VMEM((D_SAE,), jnp.float32),
            pltpu.VMEM((MAXNZ,), jnp.int32),
            pltpu.VMEM((MAXNZ,), jnp.float32),
            pltpu.VMEM((2, G, D_MODEL), jnp.float32),
            pltpu.VMEM((D_MODEL,), jnp.float32),
            pltpu.SemaphoreType.DMA((2,)),
            pltpu.SemaphoreType.DMA,
        ],
    )(sparse, W_dec)


def _decode_body(sparse_ref, w_ref, out_ref):
    @pl.when(pl.program_id(0) == 0)
    def _():
        out_ref[...] = jnp.zeros_like(out_ref)

    out_ref[...] += jnp.dot(sparse_ref[...], w_ref[...],
                            preferred_element_type=jnp.float32,
                            precision=jax.lax.Precision.DEFAULT)


def _loss_body(recon_p_ref, b_dec_ref, diff_ref, recon_ref, loss_ref):
    recon = recon_p_ref[...] + b_dec_ref[...]
    recon_ref[...] = recon
    r = recon - diff_ref[...]
    loss_ref[0, 0] = jnp.sum(r * r) / (B * D_MODEL)


@jax.jit
def kernel(x, W_enc, b_enc, W_dec, b_dec):
    f32 = jnp.float32

    diff = pl.pallas_call(
        _preprocess_body,
        out_shape=jax.ShapeDtypeStruct((B, D_MODEL), f32),
    )(x)

    nblk = D_SAE // F_BLK
    acts, scores = pl.pallas_call(
        _encode_body,
        grid=(nblk,),
        in_specs=[
            pl.BlockSpec((B, D_MODEL), lambda j: (0, 0)),
            pl.BlockSpec((D_MODEL, F_BLK), lambda j: (0, j)),
            pl.BlockSpec((1, F_BLK), lambda j: (0, j)),
        ],
        out_specs=[
            pl.BlockSpec((B, F_BLK), lambda j: (0, j)),
            pl.BlockSpec((B, F_BLK), lambda j: (0, j)),
        ],
        out_shape=[
            jax.ShapeDtypeStruct((B, D_SAE), f32),
            jax.ShapeDtypeStruct((B, D_SAE), f32),
        ],
    )(diff, W_enc, b_enc.reshape(1, D_SAE))

    sparse, _thresh = pl.pallas_call(
        _topk_body,
        out_shape=[
            jax.ShapeDtypeStruct((B, D_SAE), f32),
            jax.ShapeDtypeStruct((1, 1), f32),
        ],
        out_specs=[pl.BlockSpec(memory_space=pltpu.VMEM),
                   pl.BlockSpec(memory_space=pltpu.SMEM)],
    )(scores, acts)

    recon_p = _sc_decode(sparse, W_dec)

    recon, loss = pl.pallas_call(
        _loss_body,
        out_shape=[
            jax.ShapeDtypeStruct((B, D_MODEL), f32),
            jax.ShapeDtypeStruct((1, 1), f32),
        ],
        out_specs=[pl.BlockSpec(memory_space=pltpu.VMEM),
                   pl.BlockSpec(memory_space=pltpu.SMEM)],
    )(recon_p, b_dec.reshape(1, D_MODEL), diff)

    return (loss[0, 0], sparse, diff, recon)


# speedup vs baseline: 1.2330x; 1.2330x over previous
import jax
import jax.numpy as jnp
from jax import lax
from jax.experimental import pallas as pl
from jax.experimental.pallas import tpu as pltpu
from jax.experimental.pallas import tpu_sc as plsc

D_MODEL = 2048
D_SAE = 32768
K = 64
B = 64
EPS = 1e-08
KB = K * B

F_BLK = 2048


def _preprocess_body(x_ref, diff_ref):
    x = x_ref[...]
    d0 = x[:, D_MODEL:] - x[:, :D_MODEL]
    mu = jnp.mean(d0, axis=0, keepdims=True)
    c = d0 - mu
    norms = jnp.sqrt(jnp.sum(c * c, axis=1, keepdims=True))
    scale = jnp.mean(norms)
    diff_ref[...] = c / (scale + EPS)


def _encode_body(diff_ref, w_ref, b_ref, acts_ref, scores_ref):
    w = w_ref[...]
    h = jnp.dot(diff_ref[...], w, preferred_element_type=jnp.float32,
                precision=jax.lax.Precision.DEFAULT)
    acts = jnp.maximum(h + b_ref[...], 0.0)
    norms = jnp.sqrt(jnp.sum(w * w, axis=0, keepdims=True))
    acts_ref[...] = acts
    scores_ref[...] = acts * norms


def _topk_body(scores_ref, acts_ref, sparse_ref, thresh_ref):
    scores = scores_ref[...]
    smax = jnp.max(scores)
    hi0 = jax.lax.bitcast_convert_type(smax, jnp.int32) + 1

    def step(_, carry):
        lo, hi = carry
        mid = lo + (hi - lo) // 2
        t = jax.lax.bitcast_convert_type(mid, jnp.float32)
        cnt = jnp.sum((scores >= t).astype(jnp.int32))
        big = cnt >= KB
        return (jnp.where(big, mid, lo), jnp.where(big, hi, mid))

    lo, _ = jax.lax.fori_loop(0, 31, step, (jnp.int32(0), hi0))
    t = jax.lax.bitcast_convert_type(lo, jnp.float32)
    thresh_ref[0, 0] = t
    sparse_ref[...] = jnp.where(scores >= t, acts_ref[...], 0.0)


L = 16
UNROLL = 8
G = 16
MAXNZ = K * B
ROWS_PER_W = 2


def _sc_decode_tec(sparse_hbm, wdec_hbm, out_hbm, row_v,
                   idx_v, val_v, gbuf_v, acc_v, sem, osem):
    info = plsc.get_sparse_core_info()
    nc = info.num_cores
    wid = lax.axis_index("s") * nc + lax.axis_index("c")
    lanes = lax.iota(jnp.int32, L)
    zero16 = jnp.zeros((L,), jnp.float32)

    def do_row(r, _):
        row = wid * ROWS_PER_W + r
        pltpu.sync_copy(sparse_hbm.at[row], row_v)

        with jax.named_scope("sc_scan"):

            @plsc.parallel_loop(0, D_SAE // L, unroll=UNROLL,
                                carry=jnp.zeros((L,), jnp.int32))
            def scan_chunk(j, curv):
                v = row_v[pl.ds(j * L, L)]
                m = v != 0.0
                inc = plsc.cumsum(m.astype(jnp.int32))
                pos = curv + inc - 1
                plsc.store_scatter(idx_v, [pos], j * L + lanes, mask=m)
                plsc.store_scatter(val_v, [pos], v, mask=m)
                return curv + plsc.all_reduce_population_count(m)

            curv = scan_chunk
        cursor = curv[0]

        pad_base = (cursor // L) * L
        keep = lanes < (cursor - pad_base)
        val_v[pl.ds(pad_base, L)] = jnp.where(keep,
                                              val_v[pl.ds(pad_base, L)], 0.0)
        idx_v[pl.ds(pad_base, L)] = jnp.where(keep,
                                              idx_v[pl.ds(pad_base, L)], 0)

        for c in range(D_MODEL // L):
            acc_v[pl.ds(c * L, L)] = zero16

        nb = (cursor + G - 1) // G

        def start_gather(b, p):
            pltpu.make_async_copy(wdec_hbm.at[idx_v.at[pl.ds(b * G, G)]],
                                  gbuf_v.at[p], sem.at[p]).start()

        @pl.when(nb > 0)
        def _():
            start_gather(0, 0)

        def do_batch(b, _):
            p = lax.rem(b, 2)

            @pl.when(b + 1 < nb)
            def _():
                start_gather(b + 1, 1 - p)

            pltpu.make_async_copy(wdec_hbm.at[idx_v.at[pl.ds(b * G, G)]],
                                  gbuf_v.at[p], sem.at[p]).wait()
            vals = [plsc.load_gather(val_v,
                                     [jnp.full((L,), b * G + g, jnp.int32)])
                    for g in range(G)]

            @plsc.parallel_loop(0, D_MODEL // L, unroll=4)
            def do_col(c):
                a0 = vals[0] * gbuf_v[p, 0, pl.ds(c * L, L)]
                a1 = vals[1] * gbuf_v[p, 1, pl.ds(c * L, L)]
                for g in range(2, G, 2):
                    a0 = a0 + vals[g] * gbuf_v[p, g, pl.ds(c * L, L)]
                    a1 = a1 + vals[g + 1] * gbuf_v[p, g + 1, pl.ds(c * L, L)]
                acc_v[pl.ds(c * L, L)] = acc_v[pl.ds(c * L, L)] + a0 + a1

            return 0

        with jax.named_scope("sc_gather"):
            lax.fori_loop(0, nb, do_batch, 0)
        pltpu.async_copy(acc_v, out_hbm.at[row], osem).wait()
        return 0

    lax.fori_loop(0, ROWS_PER_W, do_row, 0)


def _sc_decode(sparse, W_dec):
    mesh = plsc.VectorSubcoreMesh(core_axis_name="c", subcore_axis_name="s")
    return pl.kernel(
        _sc_decode_tec,
        out_type=jax.ShapeDtypeStruct((B, D_MODEL), jnp.float32),
        mesh=mesh,
        compiler_params=pltpu.CompilerParams(needs_layout_passes=False),
        scratch_types=[
            pltpu.VMEM((D_SAE,), jnp.float32),
            pltpu.VMEM((MAXNZ,), jnp.int32),
            pltpu.VMEM((MAXNZ,), jnp.float32),
            pltpu.VMEM((2, G, D_MODEL), jnp.float32),
            pltpu.VMEM((D_MODEL,), jnp.float32),
            pltpu.SemaphoreType.DMA((2,)),
            pltpu.SemaphoreType.DMA,
        ],
    )(sparse, W_dec)


def _decode_body(sparse_ref, w_ref, out_ref):
    @pl.when(pl.program_id(0) == 0)
    def _():
        out_ref[...] = jnp.zeros_like(out_ref)

    out_ref[...] += jnp.dot(sparse_ref[...], w_ref[...],
                            preferred_element_type=jnp.float32,
                            precision=jax.lax.Precision.DEFAULT)


def _loss_body(recon_p_ref, b_dec_ref, diff_ref, recon_ref, loss_ref):
    recon = recon_p_ref[...] + b_dec_ref[...]
    recon_ref[...] = recon
    r = recon - diff_ref[...]
    loss_ref[0, 0] = jnp.sum(r * r) / (B * D_MODEL)


@jax.jit
def kernel(x, W_enc, b_enc, W_dec, b_dec):
    f32 = jnp.float32

    diff = pl.pallas_call(
        _preprocess_body,
        out_shape=jax.ShapeDtypeStruct((B, D_MODEL), f32),
    )(x)

    nblk = D_SAE // F_BLK
    acts, scores = pl.pallas_call(
        _encode_body,
        grid=(nblk,),
        in_specs=[
            pl.BlockSpec((B, D_MODEL), lambda j: (0, 0)),
            pl.BlockSpec((D_MODEL, F_BLK), lambda j: (0, j)),
            pl.BlockSpec((1, F_BLK), lambda j: (0, j)),
        ],
        out_specs=[
            pl.BlockSpec((B, F_BLK), lambda j: (0, j)),
            pl.BlockSpec((B, F_BLK), lambda j: (0, j)),
        ],
        out_shape=[
            jax.ShapeDtypeStruct((B, D_SAE), f32),
            jax.ShapeDtypeStruct((B, D_SAE), f32),
        ],
    )(diff, W_enc, b_enc.reshape(1, D_SAE))

    sparse, _thresh = pl.pallas_call(
        _topk_body,
        out_shape=[
            jax.ShapeDtypeStruct((B, D_SAE), f32),
            jax.ShapeDtypeStruct((1, 1), f32),
        ],
        out_specs=[pl.BlockSpec(memory_space=pltpu.VMEM),
                   pl.BlockSpec(memory_space=pltpu.SMEM)],
    )(scores, acts)

    recon_p = _sc_decode(sparse, W_dec)

    recon, loss = pl.pallas_call(
        _loss_body,
        out_shape=[
            jax.ShapeDtypeStruct((B, D_MODEL), f32),
            jax.ShapeDtypeStruct((1, 1), f32),
        ],
        out_specs=[pl.BlockSpec(memory_space=pltpu.VMEM),
                   pl.BlockSpec(memory_space=pltpu.SMEM)],
    )(recon_p, b_dec.reshape(1, D_MODEL), diff)

    return (loss[0, 0], sparse, diff, recon)
